# baseline (speedup 1.0000x reference)
import functools

import jax
import jax.numpy as jnp
from jax import lax
from jax.experimental import pallas as pl
from jax.experimental.pallas import tpu as pltpu


def _outer(C, nper, x_hbm, pe_hbm, out_hbm, pev, psem):
    D = x_hbm.shape[1]
    n = x_hbm.shape[0] // C

    for c in range(nper):
        pltpu.async_copy(
            pe_hbm.at[pl.ds(c * C, C)], pev.at[pl.ds(c * C, C)], psem.at[c]
        )

    def inner(idxs, x_ref, o_ref):
        i = idxs[0]
        off = lax.rem(i, nper) * C

        @pl.when(i < nper)
        def _wait_pe():
            pltpu.make_async_copy(
                pe_hbm.at[pl.ds(off, C)], pev.at[pl.ds(off, C)], psem.at[i]
            ).wait()

        o_ref[...] = x_ref[...] + pev[pl.ds(off, C), :]

    pltpu.emit_pipeline(
        inner,
        grid=(n,),
        in_specs=[pl.BlockSpec((C, D), lambda i: (i, 0))],
        out_specs=[pl.BlockSpec((C, D), lambda i: (i, 0))],
        _explicit_indices=True,
    )(x_hbm, out_hbm)


def kernel(x, pe):
    B, S, D = x.shape
    C = 512
    nper = S // C
    x2 = x.reshape(B * S, D)
    out = pl.pallas_call(
        functools.partial(_outer, C, nper),
        in_specs=[
            pl.BlockSpec(memory_space=pltpu.HBM),
            pl.BlockSpec(memory_space=pltpu.HBM),
        ],
        out_specs=pl.BlockSpec(memory_space=pltpu.HBM),
        out_shape=jax.ShapeDtypeStruct((B * S, D), x.dtype),
        scratch_shapes=[
            pltpu.VMEM((S, D), jnp.float32),
            pltpu.SemaphoreType.DMA((nper,)),
        ],
    )(x2, pe[:S])
    return out.reshape(B, S, D)


# speedup vs baseline: 1.1246x; 1.1246x over previous
import functools

import jax
import jax.numpy as jnp
from jax import lax
from jax.experimental import pallas as pl
from jax.experimental.pallas import tpu as pltpu


def _outer(C, nper, x_hbm, pe_hbm, out_hbm, pev, psem):
    D = x_hbm.shape[1]
    n = x_hbm.shape[0] // C

    for c in range(nper):
        pltpu.async_copy(
            pe_hbm.at[pl.ds(c * C, C)], pev.at[pl.ds(c * C, C)], psem.at[c]
        )

    def inner(idxs, x_ref, o_ref):
        i = idxs[0]
        off = lax.rem(i, nper) * C

        @pl.when(i < nper)
        def _wait_pe():
            pltpu.make_async_copy(
                pe_hbm.at[pl.ds(off, C)], pev.at[pl.ds(off, C)], psem.at[i]
            ).wait()

        o_ref[...] = x_ref[...] + pev[pl.ds(off, C), :]

    pltpu.emit_pipeline(
        inner,
        grid=(n,),
        in_specs=[pl.BlockSpec((C, D), lambda i: (i, 0))],
        out_specs=[pl.BlockSpec((C, D), lambda i: (i, 0))],
        _explicit_indices=True,
    )(x_hbm, out_hbm)


def kernel(x, pe):
    B, S, D = x.shape
    C = 2048
    nper = S // C
    x2 = x.reshape(B * S, D)
    out = pl.pallas_call(
        functools.partial(_outer, C, nper),
        in_specs=[
            pl.BlockSpec(memory_space=pltpu.HBM),
            pl.BlockSpec(memory_space=pltpu.HBM),
        ],
        out_specs=pl.BlockSpec(memory_space=pltpu.HBM),
        out_shape=jax.ShapeDtypeStruct((B * S, D), x.dtype),
        scratch_shapes=[
            pltpu.VMEM((S, D), jnp.float32),
            pltpu.SemaphoreType.DMA((nper,)),
        ],
    )(x2, pe[:S])
    return out.reshape(B, S, D)
